# baseline (speedup 1.0000x reference)
import functools

import jax
import jax.numpy as jnp
from jax import lax
from jax.experimental import pallas as pl
from jax.experimental.pallas import tpu as pltpu
from jax.experimental.pallas import tpu_sc as plsc

NUM_ROWS = 4096 * 50
DIM = 128
NC = 2
NS = 16
NW = NC * NS
SC_ROWS = 163840
B_PER_W = SC_ROWS // NW
CHUNK = 128
N_CHUNKS = B_PER_W // CHUNK
LANES = 16
NBUF = 5
LOOKAHEAD = 3
N_GROUPS = N_CHUNKS // NBUF


def _make_lookup_kernel():
    mesh = plsc.VectorSubcoreMesh(core_axis_name="c", subcore_axis_name="s")

    @functools.partial(
        pl.kernel,
        mesh=mesh,
        out_type=jax.ShapeDtypeStruct((SC_ROWS, DIM), jnp.float32),
        scratch_types=[
            pltpu.VMEM((N_CHUNKS, CHUNK), jnp.int32),
            pltpu.VMEM((DIM,), jnp.float32),
            pltpu.VMEM((NBUF, CHUNK, DIM), jnp.float32),
            pltpu.SemaphoreType.DMA((NBUF,)),
            pltpu.SemaphoreType.DMA((NBUF,)),
        ],
    )
    def k(x_hbm, table_hbm, mask_hbm, out_hbm, idx_v, mask_v, rows_v, gsem, osem):
        wid = lax.axis_index("s") * NC + lax.axis_index("c")
        base = wid * B_PER_W
        pltpu.sync_copy(x_hbm.at[wid], idx_v)
        pltpu.sync_copy(mask_hbm, mask_v)
        m = [mask_v[pl.ds(l * LANES, LANES)] for l in range(DIM // LANES)]

        def start_gather(c, b):
            pltpu.async_copy(table_hbm.at[idx_v.at[c]], rows_v.at[b], gsem.at[b])

        def wait_gather(c, b):
            pltpu.make_async_copy(
                table_hbm.at[idx_v.at[c]], rows_v.at[b], gsem.at[b]).wait()

        def start_write(c, b):
            pltpu.async_copy(
                rows_v.at[b], out_hbm.at[pl.ds(base + c * CHUNK, CHUNK)],
                osem.at[b])

        def wait_write(c, b):
            pltpu.make_async_copy(
                rows_v.at[b], out_hbm.at[pl.ds(base + c * CHUNK, CHUNK)],
                osem.at[b]).wait()

        def multiply(b):
            buf = rows_v.at[b]

            def row_body(r, carry):
                for l in range(DIM // LANES):
                    sl = pl.ds(l * LANES, LANES)
                    buf[r, sl] = buf[r, sl] * m[l]
                return carry

            lax.fori_loop(0, CHUNK, row_body, 0)

        for c in range(LOOKAHEAD):
            start_gather(c, c % NBUF)

        for b in range(NBUF):
            c = b
            wait_gather(c, b)
            multiply(b)
            start_write(c, b)
            nc = c + LOOKAHEAD
            nb = nc % NBUF
            if nc >= NBUF:
                wait_write(nc - NBUF, nb)
            start_gather(nc, nb)

        def group_body(g, carry):
            c0 = g * NBUF
            for b in range(NBUF):
                c = c0 + b
                wait_gather(c, b)
                multiply(b)
                start_write(c, b)
                nc = c + LOOKAHEAD
                nb = (b + LOOKAHEAD) % NBUF

                @pl.when(nc < N_CHUNKS)
                def _():
                    wait_write(nc - NBUF, nb)
                    start_gather(nc, nb)

            return carry

        lax.fori_loop(1, N_GROUPS, group_body, 0)

        for b in range(NBUF):
            wait_write(N_CHUNKS - NBUF + b, b)

    return k


_lookup = _make_lookup_kernel()


@jax.jit
def kernel(x, weight, band_mask):
    xf = x.reshape(-1).astype(jnp.int32)
    x_head = xf[:SC_ROWS].reshape(NW, N_CHUNKS, CHUNK)
    out_head = _lookup(x_head, weight, band_mask)
    masked = weight * band_mask[None, :]
    out_tail = jnp.take(masked, xf[SC_ROWS:], axis=0)
    out = jnp.concatenate([out_head, out_tail], axis=0)
    return out.reshape(x.shape[0], x.shape[1], DIM)


# speedup vs baseline: 1.3801x; 1.3801x over previous
import functools

import jax
import jax.numpy as jnp
from jax import lax
from jax.experimental import pallas as pl
from jax.experimental.pallas import tpu as pltpu
from jax.experimental.pallas import tpu_sc as plsc

NUM_ROWS = 4096 * 50
DIM = 128
NC = 2
NS = 16
NW = NC * NS
B_PER_W = NUM_ROWS // NW
CHUNK = 128
N_CHUNKS = B_PER_W // CHUNK
SLOT = 2 * CHUNK
N_SLOTS = B_PER_W // SLOT
LANES = 16
NBUF = 3
LOOKAHEAD = 2


def _make_lookup_kernel():
    mesh = plsc.VectorSubcoreMesh(core_axis_name="c", subcore_axis_name="s")

    @functools.partial(
        pl.kernel,
        mesh=mesh,
        out_type=jax.ShapeDtypeStruct((NUM_ROWS, DIM), jnp.float32),
        scratch_types=[
            pltpu.VMEM((N_CHUNKS, CHUNK), jnp.int32),
            pltpu.VMEM((DIM,), jnp.float32),
            pltpu.VMEM((NBUF, SLOT, DIM), jnp.float32),
            pltpu.SemaphoreType.DMA((NBUF,)),
            pltpu.SemaphoreType.DMA((NBUF,)),
        ],
    )
    def k(x_hbm, table_hbm, mask_hbm, out_hbm, idx_v, mask_v, rows_v, gsem, osem):
        wid = lax.axis_index("s") * NC + lax.axis_index("c")
        base = wid * B_PER_W
        pltpu.sync_copy(x_hbm.at[wid], idx_v)
        pltpu.sync_copy(mask_hbm, mask_v)
        m = [mask_v[pl.ds(l * LANES, LANES)] for l in range(DIM // LANES)]

        def start_gathers(s, b):
            for j in range(2):
                pltpu.async_copy(
                    table_hbm.at[idx_v.at[2 * s + j]],
                    rows_v.at[b, pl.ds(j * CHUNK, CHUNK)], gsem.at[b])

        def wait_gathers(s, b):
            for j in range(2):
                pltpu.make_async_copy(
                    table_hbm.at[idx_v.at[2 * s + j]],
                    rows_v.at[b, pl.ds(j * CHUNK, CHUNK)], gsem.at[b]).wait()

        def start_write(s, b):
            pltpu.async_copy(
                rows_v.at[b], out_hbm.at[pl.ds(base + s * SLOT, SLOT)],
                osem.at[b])

        def wait_write(s, b):
            pltpu.make_async_copy(
                rows_v.at[b], out_hbm.at[pl.ds(base + s * SLOT, SLOT)],
                osem.at[b]).wait()

        def multiply(b):
            buf = rows_v.at[b]

            def row_body(r, carry):
                for l in range(DIM // LANES):
                    sl = pl.ds(l * LANES, LANES)
                    buf[r, sl] = buf[r, sl] * m[l]
                return carry

            lax.fori_loop(0, SLOT, row_body, 0)

        def slot_body(s, b, first_group):
            wait_gathers(s, b)
            multiply(b)
            start_write(s, b)
            ns = s + LOOKAHEAD
            nb = (b + LOOKAHEAD) % NBUF
            if first_group:
                if ns >= NBUF:
                    wait_write(ns - NBUF, nb)
                start_gathers(ns, nb)
            else:
                @pl.when(ns < N_SLOTS)
                def _():
                    wait_write(ns - NBUF, nb)
                    start_gathers(ns, nb)

        for s in range(LOOKAHEAD):
            start_gathers(s, s % NBUF)

        for b in range(NBUF):
            slot_body(b, b, True)

        def group_body(g, carry):
            s0 = g * NBUF
            for b in range(NBUF):
                slot_body(s0 + b, b, False)
            return carry

        lax.fori_loop(1, (N_SLOTS - 1) // NBUF, group_body, 0)

        wait_gathers(N_SLOTS - 1, 0)
        multiply(0)
        start_write(N_SLOTS - 1, 0)

        for s in range(N_SLOTS - NBUF, N_SLOTS):
            wait_write(s, s % NBUF)

    return k


_lookup = _make_lookup_kernel()


@jax.jit
def kernel(x, weight, band_mask):
    x_flat = x.reshape(NW, N_CHUNKS, CHUNK).astype(jnp.int32)
    out = _lookup(x_flat, weight, band_mask)
    return out.reshape(x.shape[0], x.shape[1], DIM)


# speedup vs baseline: 1.3851x; 1.0036x over previous
import functools

import jax
import jax.numpy as jnp
from jax import lax
from jax.experimental import pallas as pl
from jax.experimental.pallas import tpu as pltpu
from jax.experimental.pallas import tpu_sc as plsc

NUM_ROWS = 4096 * 50
DIM = 128
NC = 2
NS = 16
NW = NC * NS
B_PER_W = NUM_ROWS // NW
CHUNK = 128
N_CHUNKS = B_PER_W // CHUNK
LANES = 16
NBUF = 5
LOOKAHEAD = 3
N_GROUPS = N_CHUNKS // NBUF


def _make_lookup_kernel():
    mesh = plsc.VectorSubcoreMesh(core_axis_name="c", subcore_axis_name="s")

    @functools.partial(
        pl.kernel,
        mesh=mesh,
        out_type=jax.ShapeDtypeStruct((NUM_ROWS, DIM), jnp.float32),
        scratch_types=[
            pltpu.VMEM((N_CHUNKS, CHUNK), jnp.int32),
            pltpu.VMEM((DIM,), jnp.float32),
            pltpu.VMEM((NBUF, CHUNK, DIM), jnp.float32),
            pltpu.SemaphoreType.DMA((NBUF,)),
            pltpu.SemaphoreType.DMA((NBUF,)),
        ],
    )
    def k(x_hbm, table_hbm, mask_hbm, out_hbm, idx_v, mask_v, rows_v, gsem, osem):
        wid = lax.axis_index("s") * NC + lax.axis_index("c")
        base = wid * B_PER_W
        pltpu.sync_copy(x_hbm.at[wid], idx_v)
        pltpu.sync_copy(mask_hbm, mask_v)
        m = [mask_v[pl.ds(l * LANES, LANES)] for l in range(DIM // LANES)]

        def start_gather(c, b):
            pltpu.async_copy(table_hbm.at[idx_v.at[c]], rows_v.at[b], gsem.at[b])

        def wait_gather(c, b):
            pltpu.make_async_copy(
                table_hbm.at[idx_v.at[c]], rows_v.at[b], gsem.at[b]).wait()

        def start_write(c, b):
            pltpu.async_copy(
                rows_v.at[b], out_hbm.at[pl.ds(base + c * CHUNK, CHUNK)],
                osem.at[b])

        def wait_write(c, b):
            pltpu.make_async_copy(
                rows_v.at[b], out_hbm.at[pl.ds(base + c * CHUNK, CHUNK)],
                osem.at[b]).wait()

        def multiply(b):
            buf = rows_v.at[b]

            def row_body(r, carry):
                for l in range(DIM // LANES):
                    sl = pl.ds(l * LANES, LANES)
                    buf[r, sl] = buf[r, sl] * m[l]
                return carry

            lax.fori_loop(0, CHUNK, row_body, 0)

        for c in range(LOOKAHEAD):
            start_gather(c, c % NBUF)

        for b in range(NBUF):
            c = b
            wait_gather(c, b)
            multiply(b)
            start_write(c, b)
            nc = c + LOOKAHEAD
            nb = nc % NBUF
            if nc >= NBUF:
                wait_write(nc - NBUF, nb)
            start_gather(nc, nb)

        def group_body(g, carry):
            c0 = g * NBUF
            for b in range(NBUF):
                c = c0 + b
                wait_gather(c, b)
                multiply(b)
                start_write(c, b)
                nc = c + LOOKAHEAD
                nb = (b + LOOKAHEAD) % NBUF

                @pl.when(nc < N_CHUNKS)
                def _():
                    wait_write(nc - NBUF, nb)
                    start_gather(nc, nb)

            return carry

        lax.fori_loop(1, N_GROUPS, group_body, 0)

        for b in range(NBUF):
            wait_write(N_CHUNKS - NBUF + b, b)

    return k


_lookup = _make_lookup_kernel()


@jax.jit
def kernel(x, weight, band_mask):
    x_flat = x.reshape(NW, N_CHUNKS, CHUNK).astype(jnp.int32)
    out = _lookup(x_flat, weight, band_mask)
    return out.reshape(x.shape[0], x.shape[1], DIM)
